# baseline (speedup 1.0000x reference)
import functools

import jax
import jax.numpy as jnp
from jax import lax
from jax.experimental import pallas as pl
from jax.experimental.pallas import tpu as pltpu
from jax.experimental.pallas import tpu_sc as plsc

_N = 1000
_B = 16384
_L = 16
_NPAD = 1024
_NSUB = 16
_BPW = _B // _NSUB


def _lane_allreduce(v, op):
    dnums = lax.GatherDimensionNumbers(
        offset_dims=(), collapsed_slice_dims=(0,), start_index_map=(0,))
    for sh in (8, 4, 2, 1):
        perm = lax.iota(jnp.int32, _L) ^ sh
        shuf = lax.gather(v, perm[:, None], dimension_numbers=dnums,
                          slice_sizes=(1,),
                          mode=lax.GatherScatterMode.PROMISE_IN_BOUNDS)
        v = op(v, shuf)
    return v


def _sc_body(loss_hbm, id_hbm, w_hbm, out_hbm,
             wexp_v, loss_v, id_v, red_v, stage_v, part_sh,
             sem_w, sem_l, sem_i):
    cid = lax.axis_index("c")
    sid = lax.axis_index("s")

    base = sid * _BPW
    cp_w = pltpu.async_copy(w_hbm, wexp_v.at[pl.ds(0, _N)], sem_w)
    cp_l = pltpu.async_copy(loss_hbm.at[pl.ds(base, _BPW)], loss_v, sem_l)
    cp_i = pltpu.async_copy(id_hbm.at[pl.ds(base, _BPW)], id_v, sem_i)
    cp_w.wait()

    neg = jnp.full((_L,), -jnp.inf, jnp.float32)
    tail = wexp_v[pl.ds(_NPAD - 2 * _L, _L)]
    lane = lax.iota(jnp.int32, _L)
    wexp_v[pl.ds(_NPAD - 2 * _L, _L)] = jnp.where(lane < (_N % _L), tail, neg)
    wexp_v[pl.ds(_NPAD - _L, _L)] = neg

    nv = _NPAD // _L

    @plsc.parallel_loop(0, nv, step=4, unroll=4, carry=(neg, neg, neg, neg))
    def ms(j, c):
        return tuple(
            jnp.maximum(c[k], wexp_v[pl.ds((j + k) * _L, _L)])
            for k in range(4))
    m = jnp.maximum(jnp.maximum(ms[0], ms[1]), jnp.maximum(ms[2], ms[3]))
    wmax = _lane_allreduce(m, jnp.maximum)

    zero = jnp.zeros((_L,), jnp.float32)

    @plsc.parallel_loop(0, nv, step=4, unroll=4, carry=(zero,) * 4)
    def ss(j, c):
        out = []
        for k in range(4):
            e = jnp.exp(wexp_v[pl.ds((j + k) * _L, _L)] - wmax)
            wexp_v[pl.ds((j + k) * _L, _L)] = e
            out.append(c[k] + e)
        return tuple(out)
    ssum = (ss[0] + ss[1]) + (ss[2] + ss[3])
    s_total = _lane_allreduce(ssum, jnp.add)

    cp_l.wait()
    cp_i.wait()

    @plsc.parallel_loop(0, _BPW // _L, step=4, unroll=4, carry=(zero,) * 4)
    def accs(j, c):
        out = []
        for k in range(4):
            idx = id_v[pl.ds((j + k) * _L, _L)]
            g = plsc.load_gather(wexp_v, [idx])
            out.append(c[k] + g * loss_v[pl.ds((j + k) * _L, _L)])
        return tuple(out)
    acc = (accs[0] + accs[1]) + (accs[2] + accs[3])

    stage_v[...] = acc
    pltpu.sync_copy(stage_v, part_sh.at[pl.ds(sid * _L, _L)])
    plsc.subcore_barrier()

    @pl.when(jnp.logical_and(sid == 0, cid == 0))
    def _finalize():
        pltpu.sync_copy(part_sh, red_v)
        tot = red_v[pl.ds(0, _L)]
        for j in range(1, _NSUB):
            tot = tot + red_v[pl.ds(j * _L, _L)]
        total = _lane_allreduce(tot, jnp.add)
        final = total * (jnp.float32(_N) / jnp.float32(_B)) / s_total
        stage_v[...] = final
        pltpu.sync_copy(stage_v.at[pl.ds(0, 1)], out_hbm)


@jax.jit
def _sc_call(loss, ls_id, w_pad):
    mesh = plsc.VectorSubcoreMesh(core_axis_name="c", subcore_axis_name="s",
                                  num_cores=1)
    kfn = pl.kernel(
        _sc_body,
        out_type=jax.ShapeDtypeStruct((1,), jnp.float32),
        mesh=mesh,
        compiler_params=pltpu.CompilerParams(needs_layout_passes=False),
        scratch_types=[
            pltpu.VMEM((_NPAD,), jnp.float32),
            pltpu.VMEM((_BPW,), jnp.float32),
            pltpu.VMEM((_BPW,), jnp.int32),
            pltpu.VMEM((_NSUB * _L,), jnp.float32),
            pltpu.VMEM((_L,), jnp.float32),
            pltpu.VMEM_SHARED((_NSUB * _L,), jnp.float32),
            pltpu.SemaphoreType.DMA,
            pltpu.SemaphoreType.DMA,
            pltpu.SemaphoreType.DMA,
        ],
    )
    return kfn(loss, ls_id, w_pad)


def kernel(loss, ls_id, weight):
    out = _sc_call(loss.astype(jnp.float32), ls_id.astype(jnp.int32),
                   weight.astype(jnp.float32))
    return out.reshape(())


# speedup vs baseline: 1.0115x; 1.0115x over previous
import functools

import jax
import jax.numpy as jnp
from jax import lax
from jax.experimental import pallas as pl
from jax.experimental.pallas import tpu as pltpu
from jax.experimental.pallas import tpu_sc as plsc

_N = 1000
_B = 16384
_L = 16
_NPAD = 1024
_NSUB = 16
_BPW = _B // _NSUB


def _lane_allreduce(v, op):
    dnums = lax.GatherDimensionNumbers(
        offset_dims=(), collapsed_slice_dims=(0,), start_index_map=(0,))
    for sh in (8, 4, 2, 1):
        perm = lax.iota(jnp.int32, _L) ^ sh
        shuf = lax.gather(v, perm[:, None], dimension_numbers=dnums,
                          slice_sizes=(1,),
                          mode=lax.GatherScatterMode.PROMISE_IN_BOUNDS)
        v = op(v, shuf)
    return v


def _sc_body(loss_hbm, id_hbm, w_hbm, out_hbm,
             wexp_v, loss_v, id_v, red_v, stage_v, part_sh,
             sem_w, sem_l, sem_i):
    cid = lax.axis_index("c")
    sid = lax.axis_index("s")

    base = sid * _BPW
    cp_w = pltpu.async_copy(w_hbm, wexp_v.at[pl.ds(0, _N)], sem_w)
    cp_l = pltpu.async_copy(loss_hbm.at[pl.ds(base, _BPW)], loss_v, sem_l)
    cp_i = pltpu.async_copy(id_hbm.at[pl.ds(base, _BPW)], id_v, sem_i)
    cp_w.wait()

    neg = jnp.full((_L,), -jnp.inf, jnp.float32)
    tail = wexp_v[pl.ds(_NPAD - 2 * _L, _L)]
    lane = lax.iota(jnp.int32, _L)
    wexp_v[pl.ds(_NPAD - 2 * _L, _L)] = jnp.where(lane < (_N % _L), tail, neg)
    wexp_v[pl.ds(_NPAD - _L, _L)] = neg

    nv = _NPAD // _L

    @plsc.parallel_loop(0, nv, step=4, unroll=2, carry=(neg, neg, neg, neg))
    def ms(j, c):
        return tuple(
            jnp.maximum(c[k], wexp_v[pl.ds((j + k) * _L, _L)])
            for k in range(4))
    m = jnp.maximum(jnp.maximum(ms[0], ms[1]), jnp.maximum(ms[2], ms[3]))
    wmax = _lane_allreduce(m, jnp.maximum)

    zero = jnp.zeros((_L,), jnp.float32)
    cp_l.wait()
    cp_i.wait()

    @plsc.parallel_loop(0, nv, step=4, unroll=2, carry=(zero,) * 8)
    def fused(j, c):
        out = []
        for k in range(4):
            e = jnp.exp(wexp_v[pl.ds((j + k) * _L, _L)] - wmax)
            out.append(c[k] + e)
        for k in range(4):
            idx = id_v[pl.ds((j + k) * _L, _L)]
            g = plsc.load_gather(wexp_v, [idx])
            ge = jnp.exp(g - wmax)
            out.append(c[4 + k] + ge * loss_v[pl.ds((j + k) * _L, _L)])
        return tuple(out)
    ssum = (fused[0] + fused[1]) + (fused[2] + fused[3])
    s_total = _lane_allreduce(ssum, jnp.add)
    acc = (fused[4] + fused[5]) + (fused[6] + fused[7])

    stage_v[...] = acc
    pltpu.sync_copy(stage_v, part_sh.at[pl.ds(sid * _L, _L)])
    plsc.subcore_barrier()

    @pl.when(jnp.logical_and(sid == 0, cid == 0))
    def _finalize():
        pltpu.sync_copy(part_sh, red_v)
        tot = red_v[pl.ds(0, _L)]
        for j in range(1, _NSUB):
            tot = tot + red_v[pl.ds(j * _L, _L)]
        total = _lane_allreduce(tot, jnp.add)
        final = total * (jnp.float32(_N) / jnp.float32(_B)) / s_total
        stage_v[...] = final
        pltpu.sync_copy(stage_v.at[pl.ds(0, 1)], out_hbm)


@jax.jit
def _sc_call(loss, ls_id, w_pad):
    mesh = plsc.VectorSubcoreMesh(core_axis_name="c", subcore_axis_name="s",
                                  num_cores=1)
    kfn = pl.kernel(
        _sc_body,
        out_type=jax.ShapeDtypeStruct((1,), jnp.float32),
        mesh=mesh,
        compiler_params=pltpu.CompilerParams(needs_layout_passes=False),
        scratch_types=[
            pltpu.VMEM((_NPAD,), jnp.float32),
            pltpu.VMEM((_BPW,), jnp.float32),
            pltpu.VMEM((_BPW,), jnp.int32),
            pltpu.VMEM((_NSUB * _L,), jnp.float32),
            pltpu.VMEM((_L,), jnp.float32),
            pltpu.VMEM_SHARED((_NSUB * _L,), jnp.float32),
            pltpu.SemaphoreType.DMA,
            pltpu.SemaphoreType.DMA,
            pltpu.SemaphoreType.DMA,
        ],
    )
    return kfn(loss, ls_id, w_pad)


def kernel(loss, ls_id, weight):
    out = _sc_call(loss.astype(jnp.float32), ls_id.astype(jnp.int32),
                   weight.astype(jnp.float32))
    return out.reshape(())


# speedup vs baseline: 1.0173x; 1.0058x over previous
import functools

import jax
import jax.numpy as jnp
from jax import lax
from jax.experimental import pallas as pl
from jax.experimental.pallas import tpu as pltpu
from jax.experimental.pallas import tpu_sc as plsc

_N = 1000
_B = 16384
_L = 16
_NPAD = 1024
_NSUB = 16
_BPW = _B // _NSUB


def _lane_allreduce(v, op):
    dnums = lax.GatherDimensionNumbers(
        offset_dims=(), collapsed_slice_dims=(0,), start_index_map=(0,))
    for sh in (8, 4, 2, 1):
        perm = lax.iota(jnp.int32, _L) ^ sh
        shuf = lax.gather(v, perm[:, None], dimension_numbers=dnums,
                          slice_sizes=(1,),
                          mode=lax.GatherScatterMode.PROMISE_IN_BOUNDS)
        v = op(v, shuf)
    return v


def _sc_body(loss_hbm, id_hbm, w_hbm, out_hbm,
             wexp_v, loss_v, id_v, red_v, stage_v, part_sh,
             sem_w, sem_l, sem_i):
    cid = lax.axis_index("c")
    sid = lax.axis_index("s")

    base = sid * _BPW
    cp_w = pltpu.async_copy(w_hbm, wexp_v.at[pl.ds(0, _N)], sem_w)
    cp_l = pltpu.async_copy(loss_hbm.at[pl.ds(base, _BPW)], loss_v, sem_l)
    cp_i = pltpu.async_copy(id_hbm.at[pl.ds(base, _BPW)], id_v, sem_i)

    zero = jnp.zeros((_L,), jnp.float32)

    @pl.when(sid == 0)
    def _zero_acc():
        stage_v[...] = zero
        pltpu.sync_copy(stage_v, part_sh)
    plsc.subcore_barrier()

    cp_w.wait()

    neg = jnp.full((_L,), -jnp.inf, jnp.float32)
    tail = wexp_v[pl.ds(_NPAD - 2 * _L, _L)]
    lane = lax.iota(jnp.int32, _L)
    wexp_v[pl.ds(_NPAD - 2 * _L, _L)] = jnp.where(lane < (_N % _L), tail, neg)
    wexp_v[pl.ds(_NPAD - _L, _L)] = neg

    nv = _NPAD // _L

    @plsc.parallel_loop(0, nv, step=4, unroll=2, carry=(neg, neg, neg, neg))
    def ms(j, c):
        return tuple(
            jnp.maximum(c[k], wexp_v[pl.ds((j + k) * _L, _L)])
            for k in range(4))
    m = jnp.maximum(jnp.maximum(ms[0], ms[1]), jnp.maximum(ms[2], ms[3]))
    wmax = _lane_allreduce(m, jnp.maximum)

    cp_l.wait()
    cp_i.wait()

    @plsc.parallel_loop(0, nv, step=4, unroll=2, carry=(zero,) * 8)
    def fused(j, c):
        out = []
        for k in range(4):
            e = jnp.exp(wexp_v[pl.ds((j + k) * _L, _L)] - wmax)
            out.append(c[k] + e)
        for k in range(4):
            idx = id_v[pl.ds((j + k) * _L, _L)]
            g = plsc.load_gather(wexp_v, [idx])
            ge = jnp.exp(g - wmax)
            out.append(c[4 + k] + ge * loss_v[pl.ds((j + k) * _L, _L)])
        return tuple(out)
    ssum = (fused[0] + fused[1]) + (fused[2] + fused[3])
    s_total = _lane_allreduce(ssum, jnp.add)
    acc = (fused[4] + fused[5]) + (fused[6] + fused[7])

    stage_v[...] = acc
    pltpu.sync_copy(stage_v, part_sh.at[lax.iota(jnp.int32, _L)], add=True)
    plsc.subcore_barrier()

    @pl.when(jnp.logical_and(sid == 0, cid == 0))
    def _finalize():
        pltpu.sync_copy(part_sh, red_v)
        total = _lane_allreduce(red_v[...], jnp.add)
        final = total * (jnp.float32(_N) / jnp.float32(_B)) / s_total
        stage_v[...] = final
        pltpu.sync_copy(stage_v.at[pl.ds(0, 1)], out_hbm)


@jax.jit
def _sc_call(loss, ls_id, w_pad):
    mesh = plsc.VectorSubcoreMesh(core_axis_name="c", subcore_axis_name="s",
                                  num_cores=1)
    kfn = pl.kernel(
        _sc_body,
        out_type=jax.ShapeDtypeStruct((1,), jnp.float32),
        mesh=mesh,
        compiler_params=pltpu.CompilerParams(needs_layout_passes=False),
        scratch_types=[
            pltpu.VMEM((_NPAD,), jnp.float32),
            pltpu.VMEM((_BPW,), jnp.float32),
            pltpu.VMEM((_BPW,), jnp.int32),
            pltpu.VMEM((_L,), jnp.float32),
            pltpu.VMEM((_L,), jnp.float32),
            pltpu.VMEM_SHARED((_L,), jnp.float32),
            pltpu.SemaphoreType.DMA,
            pltpu.SemaphoreType.DMA,
            pltpu.SemaphoreType.DMA,
        ],
    )
    return kfn(loss, ls_id, w_pad)


def kernel(loss, ls_id, weight):
    out = _sc_call(loss.astype(jnp.float32), ls_id.astype(jnp.int32),
                   weight.astype(jnp.float32))
    return out.reshape(())
